# baseline (speedup 1.0000x reference)
import jax
import jax.numpy as jnp
from jax import lax
from jax.experimental import pallas as pl
from jax.experimental.pallas import tpu as pltpu

P = 32
V_BLOCK = 128


def _pointnet_block(pts_ref, cnt_ref, w1_ref, w2_ref, w3_ref, b3_ref,
                    out_ref, scr_ref):
    V = out_ref.shape[0]
    i = pl.program_id(0)
    nblk = pl.num_programs(0) - 1

    @pl.when(i < nblk)
    def compute():
        pts_t = pts_ref[...]
        h1 = lax.dot_general(pts_t, w1_ref[...],
                             dimension_numbers=(((0,), (0,)), ((), ())),
                             preferred_element_type=jnp.float32)
        h1 = jnp.maximum(h1, 0.0)
        h2 = jnp.dot(h1, w2_ref[...], preferred_element_type=jnp.float32)
        h2 = jnp.maximum(h2, 0.0)
        h3 = jnp.dot(h2, w3_ref[...], preferred_element_type=jnp.float32)
        scr_ref[i % 2] = h3.reshape(P, V, 256)

    @pl.when(i > 0)
    def epilogue():
        h3 = scr_ref[(i - 1) % 2]
        cnt = cnt_ref[...]
        cnt_b = jnp.broadcast_to(jnp.maximum(cnt, 1), (V, 256))
        mask = lax.broadcasted_iota(jnp.int32, (P, V, 256), 0) < cnt_b[None]
        m = jnp.max(jnp.where(mask, h3, jnp.float32(-1e30)), axis=0)

        z1 = jnp.maximum(w1_ref[3:4, :].astype(jnp.float32), 0.0)
        z2 = jnp.maximum(
            jnp.dot(z1, w2_ref[...], preferred_element_type=jnp.float32), 0.0)
        z3 = jnp.dot(z2, w3_ref[...], preferred_element_type=jnp.float32)

        pos = jnp.broadcast_to(cnt > 0, (V, 256))
        out_ref[...] = jnp.where(pos, m, z3) + b3_ref[...]


def kernel(Frustum_Voxel, Frustum_Voxel_num, W1, b1, W2, b2, W3, b3):
    B, H, Wd, Pp, _ = Frustum_Voxel.shape
    N = B * H * Wd
    nb = N // V_BLOCK

    fv16 = Frustum_Voxel.astype(jnp.bfloat16)
    t = fv16.reshape(nb, V_BLOCK, Pp, 3).transpose(0, 2, 1, 3)
    pts_t = t.reshape(nb * Pp * V_BLOCK, 3).T
    pts_t = jnp.concatenate(
        [pts_t, jnp.ones((1, pts_t.shape[1]), jnp.bfloat16)], axis=0)
    cnt = Frustum_Voxel_num.reshape(N, 1)

    w1a = jnp.zeros((4, 33), jnp.float32)
    w1a = w1a.at[:3, :32].set(W1).at[3, :32].set(b1).at[3, 32].set(1.0)
    w1a = w1a.astype(jnp.bfloat16)
    w2a = jnp.concatenate([W2, b2.reshape(1, 256)], axis=0)

    last = nb - 1
    feat = pl.pallas_call(
        _pointnet_block,
        grid=(nb + 1,),
        in_specs=[
            pl.BlockSpec((4, Pp * V_BLOCK), lambda i: (0, jnp.minimum(i, last))),
            pl.BlockSpec((V_BLOCK, 1), lambda i: (jnp.maximum(i - 1, 0), 0)),
            pl.BlockSpec((4, 33), lambda i: (0, 0)),
            pl.BlockSpec((33, 256), lambda i: (0, 0)),
            pl.BlockSpec((256, 256), lambda i: (0, 0)),
            pl.BlockSpec((1, 256), lambda i: (0, 0)),
        ],
        out_specs=pl.BlockSpec((V_BLOCK, 256), lambda i: (jnp.maximum(i - 1, 0), 0)),
        out_shape=jax.ShapeDtypeStruct((N, 256), jnp.float32),
        scratch_shapes=[pltpu.VMEM((2, Pp, V_BLOCK, 256), jnp.float32)],
    )(pts_t, cnt, w1a, w2a, W3, b3.reshape(1, 256))

    return feat.reshape(B, H, Wd, 256)


# speedup vs baseline: 1.0471x; 1.0471x over previous
import jax
import jax.numpy as jnp
from jax import lax
from jax.experimental import pallas as pl
from jax.experimental.pallas import tpu as pltpu

P = 32
V_BLOCK = 128


def _pointnet_block(pts_ref, cnt_ref, w1_ref, w2_ref, w3_ref, b3_ref,
                    out_ref, scr_ref):
    V = out_ref.shape[0]
    i = pl.program_id(0)

    h3p = scr_ref[(i + 1) % 2]
    cnt = cnt_ref[...]
    cnt_b = jnp.broadcast_to(jnp.maximum(cnt, 1), (V, 256))
    mask = lax.broadcasted_iota(jnp.int32, (P, V, 256), 0) < cnt_b[None]
    m = jnp.max(jnp.where(mask, h3p, jnp.float32(-1e30)), axis=0)

    z1 = jnp.maximum(w1_ref[3:4, :].astype(jnp.float32), 0.0)
    z2 = jnp.maximum(
        jnp.dot(z1, w2_ref[...], preferred_element_type=jnp.float32), 0.0)
    z3 = jnp.dot(z2, w3_ref[...], preferred_element_type=jnp.float32)

    pos = jnp.broadcast_to(cnt > 0, (V, 256))
    out_ref[...] = jnp.where(pos, m, z3) + b3_ref[...]

    pts_t = pts_ref[...]
    h1 = lax.dot_general(pts_t, w1_ref[...],
                         dimension_numbers=(((0,), (0,)), ((), ())),
                         preferred_element_type=jnp.float32)
    h1 = jnp.maximum(h1, 0.0)
    h2 = jnp.dot(h1, w2_ref[...], preferred_element_type=jnp.float32)
    h2 = jnp.maximum(h2, 0.0)
    h3 = jnp.dot(h2, w3_ref[...], preferred_element_type=jnp.float32)
    scr_ref[i % 2] = h3.reshape(P, V, 256)


def kernel(Frustum_Voxel, Frustum_Voxel_num, W1, b1, W2, b2, W3, b3):
    B, H, Wd, Pp, _ = Frustum_Voxel.shape
    N = B * H * Wd
    nb = N // V_BLOCK

    fv16 = Frustum_Voxel.astype(jnp.bfloat16)
    t = fv16.reshape(nb, V_BLOCK, Pp, 3).transpose(0, 2, 1, 3)
    pts_t = t.reshape(nb * Pp * V_BLOCK, 3).T
    pts_t = jnp.concatenate(
        [pts_t, jnp.ones((1, pts_t.shape[1]), jnp.bfloat16)], axis=0)
    cnt = Frustum_Voxel_num.reshape(N, 1)

    w1a = jnp.zeros((4, 33), jnp.float32)
    w1a = w1a.at[:3, :32].set(W1).at[3, :32].set(b1).at[3, 32].set(1.0)
    w1a = w1a.astype(jnp.bfloat16)
    w2a = jnp.concatenate([W2, b2.reshape(1, 256)], axis=0)

    last = nb - 1
    feat = pl.pallas_call(
        _pointnet_block,
        grid=(nb + 1,),
        in_specs=[
            pl.BlockSpec((4, Pp * V_BLOCK), lambda i: (0, jnp.minimum(i, last))),
            pl.BlockSpec((V_BLOCK, 1), lambda i: (jnp.maximum(i - 1, 0), 0)),
            pl.BlockSpec((4, 33), lambda i: (0, 0)),
            pl.BlockSpec((33, 256), lambda i: (0, 0)),
            pl.BlockSpec((256, 256), lambda i: (0, 0)),
            pl.BlockSpec((1, 256), lambda i: (0, 0)),
        ],
        out_specs=pl.BlockSpec((V_BLOCK, 256), lambda i: (jnp.maximum(i - 1, 0), 0)),
        out_shape=jax.ShapeDtypeStruct((N, 256), jnp.float32),
        scratch_shapes=[pltpu.VMEM((2, Pp, V_BLOCK, 256), jnp.float32)],
    )(pts_t, cnt, w1a, w2a, W3, b3.reshape(1, 256))

    return feat.reshape(B, H, Wd, 256)


# speedup vs baseline: 1.3405x; 1.2802x over previous
import jax
import jax.numpy as jnp
from jax import lax
from jax.experimental import pallas as pl

P = 32
V_BLOCK = 512


def _pointnet_block(pts_ref, cnt_ref, w1_ref, w2_ref, w3_ref, b3_ref,
                    out_ref):
    V = out_ref.shape[0]
    cnt = cnt_ref[...]
    pts_t = pts_ref[...]

    h1 = lax.dot_general(pts_t, w1_ref[...],
                         dimension_numbers=(((0,), (0,)), ((), ())),
                         preferred_element_type=jnp.float32)
    h1 = jnp.maximum(h1, 0.0)

    h13 = h1.reshape(P, V, 33)
    empty33 = jnp.broadcast_to(cnt <= 0, (V, 33))
    slab0 = lax.broadcasted_iota(jnp.int32, (P, V, 33), 0) == 0
    zero_h1 = jnp.maximum(w1_ref[3:4, :].astype(jnp.float32), 0.0)
    h13 = jnp.where(slab0 & empty33[None], zero_h1[None], h13)
    h1 = h13.reshape(P * V, 33)

    h2 = jnp.dot(h1, w2_ref[...], preferred_element_type=jnp.float32)
    h2 = jnp.maximum(h2, 0.0)
    h3 = jnp.dot(h2, w3_ref[...], preferred_element_type=jnp.float32)
    h3 = h3.reshape(P, V, 256)

    cnt_b = jnp.broadcast_to(jnp.maximum(cnt, 1), (V, 256))
    mask = lax.broadcasted_iota(jnp.int32, (P, V, 256), 0) < cnt_b[None]
    masked = jnp.where(mask, h3, jnp.float32(-1e30))
    out_ref[...] = jnp.max(masked, axis=0) + b3_ref[...]


def kernel(Frustum_Voxel, Frustum_Voxel_num, W1, b1, W2, b2, W3, b3):
    B, H, Wd, Pp, _ = Frustum_Voxel.shape
    N = B * H * Wd
    nb = N // V_BLOCK

    fv16 = Frustum_Voxel.astype(jnp.bfloat16)
    t = fv16.reshape(nb, V_BLOCK, Pp, 3).transpose(0, 2, 1, 3)
    pts_t = t.reshape(nb * Pp * V_BLOCK, 3).T
    pts_t = jnp.concatenate(
        [pts_t, jnp.ones((1, pts_t.shape[1]), jnp.bfloat16)], axis=0)
    cnt = Frustum_Voxel_num.reshape(N, 1)

    w1a = jnp.zeros((4, 33), jnp.float32)
    w1a = w1a.at[:3, :32].set(W1).at[3, :32].set(b1).at[3, 32].set(1.0)
    w1a = w1a.astype(jnp.bfloat16)
    w2a = jnp.concatenate([W2, b2.reshape(1, 256)], axis=0)

    feat = pl.pallas_call(
        _pointnet_block,
        grid=(nb,),
        in_specs=[
            pl.BlockSpec((4, Pp * V_BLOCK), lambda i: (0, i)),
            pl.BlockSpec((V_BLOCK, 1), lambda i: (i, 0)),
            pl.BlockSpec((4, 33), lambda i: (0, 0)),
            pl.BlockSpec((33, 256), lambda i: (0, 0)),
            pl.BlockSpec((256, 256), lambda i: (0, 0)),
            pl.BlockSpec((1, 256), lambda i: (0, 0)),
        ],
        out_specs=pl.BlockSpec((V_BLOCK, 256), lambda i: (i, 0)),
        out_shape=jax.ShapeDtypeStruct((N, 256), jnp.float32),
    )(pts_t, cnt, w1a, w2a, W3, b3.reshape(1, 256))

    return feat.reshape(B, H, Wd, 256)
